# baseline (speedup 1.0000x reference)
import functools

import jax
import jax.numpy as jnp
from jax import lax
from jax.experimental import pallas as pl
from jax.experimental.pallas import tpu as pltpu
from jax.experimental.pallas import tpu_sc as plsc

_NUM_CORES = 2
_NUM_SUBCORES = 16
_NUM_WORKERS = _NUM_CORES * _NUM_SUBCORES
_CHUNK = 1600


@functools.lru_cache(maxsize=None)
def _make_gather(n_rows, d):
    rows_per_w = n_rows // _NUM_WORKERS
    n_chunks = rows_per_w // _CHUNK
    mesh = plsc.VectorSubcoreMesh(core_axis_name="c", subcore_axis_name="s")

    @functools.partial(
        pl.kernel,
        mesh=mesh,
        out_type=jax.ShapeDtypeStruct((n_rows, d), jnp.float32),
        scratch_types=[
            pltpu.VMEM((rows_per_w,), jnp.int32),
            pltpu.VMEM((2, _CHUNK, d), jnp.float32),
            pltpu.SemaphoreType.DMA,
            pltpu.SemaphoreType.DMA,
            pltpu.SemaphoreType.DMA,
            pltpu.SemaphoreType.DMA,
        ],
        compiler_params=pltpu.CompilerParams(use_tc_tiling_on_sc=False),
    )
    def gather_kernel(table_hbm, idx_hbm, out_hbm, idx_v, rows_v, sg0, sg1,
                      ss0, ss1):
        wid = lax.axis_index("s") * _NUM_CORES + lax.axis_index("c")
        base = pl.multiple_of(wid * rows_per_w, 8)
        pltpu.sync_copy(idx_hbm.at[pl.ds(base, rows_per_w)], idx_v)

        sem_g = (sg0, sg1)
        sem_s = (ss0, ss1)

        def gather_start(g):
            return pltpu.async_copy(
                table_hbm.at[idx_v.at[pl.ds(g * _CHUNK, _CHUNK)]],
                rows_v.at[g % 2], sem_g[g % 2])

        def store_start(g):
            off = pl.multiple_of(base + g * _CHUNK, 8)
            return pltpu.async_copy(
                rows_v.at[g % 2], out_hbm.at[pl.ds(off, _CHUNK)],
                sem_s[g % 2])

        stores = [None] * n_chunks
        pending = gather_start(0)
        for g in range(n_chunks):
            pending.wait()
            stores[g] = store_start(g)
            if g + 1 < n_chunks:
                if g >= 1:
                    stores[g - 1].wait()
                pending = gather_start(g + 1)
        if n_chunks >= 2:
            stores[n_chunks - 2].wait()
        stores[n_chunks - 1].wait()

    return gather_kernel


def kernel(effect_id, W):
    b, h = effect_id.shape
    d = W.shape[1]
    idx = effect_id.reshape(-1).astype(jnp.int32)
    out = _make_gather(b * h, d)(W, idx)
    return out.reshape(b, h * d)
